# Initial kernel scaffold; baseline (speedup 1.0000x reference)
import jax
import jax.numpy as jnp
from jax.experimental import pallas as pl


def kernel(positions, table_0, table_1, table_2, table_3, level_weights):
    raise NotImplementedError("write your pallas kernel here")


# speedup vs baseline: 17.2784x; 17.2784x over previous
import functools

import jax
import jax.numpy as jnp
from jax import lax
from jax.experimental import pallas as pl
from jax.experimental.pallas import tpu as pltpu
from jax.experimental.pallas import tpu_sc as plsc

N = 1048576
NUM_LEVELS = 4
HASH_SIZES = (15, 17, 19, 21)
INV_CELL = (16.0, 64.0, 256.0, 1024.0)
C1 = 73856093
C2 = 19349663

NC = 2
NS = 16
NW = NC * NS
LANES = 16

B = 512
G = B // 128
PER_TILE = N // NW
CHUNKS = PER_TILE // B


def _full(v):
    return jnp.full((LANES,), v, jnp.int32)


def _body(pos_hbm, t0, t1, t2, t3, lw_hbm, out_hbm,
          pos_v, idx_v, wx_v, wy_v, rows_v, out_v, lw_v, sem):
    tables = (t0, t1, t2, t3)
    wid = lax.axis_index("s") * NC + lax.axis_index("c")
    tile_base = wid * PER_TILE

    pltpu.sync_copy(lw_hbm, lw_v)
    w = lw_v[...]
    w0 = plsc.load_gather(lw_v, [_full(0)])
    w1 = plsc.load_gather(lw_v, [_full(1)])
    w2 = plsc.load_gather(lw_v, [_full(2)])
    w3 = plsc.load_gather(lw_v, [_full(3)])
    m = jnp.maximum(jnp.maximum(w0, w1), jnp.maximum(w2, w3))
    lw_v[...] = jnp.exp(w - m)
    e = lw_v[...]
    e0 = plsc.load_gather(lw_v, [_full(0)])
    e1 = plsc.load_gather(lw_v, [_full(1)])
    e2 = plsc.load_gather(lw_v, [_full(2)])
    e3 = plsc.load_gather(lw_v, [_full(3)])
    lw_v[...] = e / ((e0 + e1) + (e2 + e3))

    iota = lax.iota(jnp.int32, LANES)

    def chunk_body(ci, carry):
        gbase = tile_base + ci * B
        pltpu.sync_copy(pos_hbm.at[pl.ds(gbase, B)], pos_v)

        def pa_body(j, c):
            r16 = pl.multiple_of(j * LANES, LANES) + iota
            px = plsc.load_gather(pos_v, [r16, _full(0)])
            py = plsc.load_gather(pos_v, [r16, _full(1)])
            gvec = jnp.full((LANES,), j >> 3, jnp.int32) + _full(0)
            colv = (j & 7) * LANES + iota
            for l in range(NUM_LEVELS):
                inv = INV_CELL[l]
                mask = (1 << HASH_SIZES[l]) - 1
                sx = px * inv
                sy = py * inv
                ix = sx.astype(jnp.int32)
                iy = sy.astype(jnp.int32)
                wx = sx - ix.astype(jnp.float32)
                wy = sy - iy.astype(jnp.float32)
                off = pl.multiple_of(j * LANES, LANES)
                wx_v[l, pl.ds(off, LANES)] = wx
                wy_v[l, pl.ds(off, LANES)] = wy
                hx0 = ix * C1
                hx1 = hx0 + C1
                hy0 = iy * C2
                hy1 = hy0 + C2
                hs = (hx0 ^ hy0, hx1 ^ hy0, hx0 ^ hy1, hx1 ^ hy1)
                for cidx in range(4):
                    cc = l * 4 + cidx
                    plsc.store_scatter(idx_v, [_full(cc), gvec, colv],
                                       hs[cidx] & mask)
            return c

        lax.fori_loop(0, B // LANES, pa_body, 0)

        copies = []
        for cc in range(4 * NUM_LEVELS):
            tbl = tables[cc // 4]
            for g in range(G):
                copies.append(pltpu.async_copy(
                    tbl.at[idx_v.at[cc, g]],
                    rows_v.at[cc, pl.ds(g * 128, 128)],
                    sem))
        for cp in copies:
            cp.wait()

        pair = iota >> 1
        parity = iota & 1

        def pc_body(i, c):
            rowv = i * 8 + pair
            for l in range(NUM_LEVELS):
                base_c = l * 4
                f00 = plsc.load_gather(rows_v, [_full(base_c), rowv, parity])
                f10 = plsc.load_gather(rows_v, [_full(base_c + 1), rowv, parity])
                f01 = plsc.load_gather(rows_v, [_full(base_c + 2), rowv, parity])
                f11 = plsc.load_gather(rows_v, [_full(base_c + 3), rowv, parity])
                wxd = plsc.load_gather(wx_v, [_full(l), rowv])
                wyd = plsc.load_gather(wy_v, [_full(l), rowv])
                wl = plsc.load_gather(lw_v, [_full(l)])
                a = f00 + wxd * (f10 - f00)
                b = f01 + wxd * (f11 - f01)
                res = a + wyd * (b - a)
                plsc.store_scatter(out_v, [rowv, _full(2 * l) + parity],
                                   wl * res)
            return c

        lax.fori_loop(0, B // 8, pc_body, 0)

        pltpu.sync_copy(out_v, out_hbm.at[pl.ds(gbase, B)])
        return carry

    lax.fori_loop(0, CHUNKS, chunk_body, 0)


_mesh = plsc.VectorSubcoreMesh(core_axis_name="c", subcore_axis_name="s")

_call = functools.partial(
    pl.kernel,
    mesh=_mesh,
    compiler_params=pltpu.CompilerParams(
        needs_layout_passes=False, use_tc_tiling_on_sc=False),
    out_type=jax.ShapeDtypeStruct((N, 8), jnp.float32),
    scratch_types=[
        pltpu.VMEM((B, 2), jnp.float32),
        pltpu.VMEM((4 * NUM_LEVELS, G, 128), jnp.int32),
        pltpu.VMEM((NUM_LEVELS, B), jnp.float32),
        pltpu.VMEM((NUM_LEVELS, B), jnp.float32),
        pltpu.VMEM((4 * NUM_LEVELS, B, 2), jnp.float32),
        pltpu.VMEM((B, 8), jnp.float32),
        pltpu.VMEM((LANES,), jnp.float32),
        pltpu.SemaphoreType.DMA,
    ],
)(_body)


@jax.jit
def kernel(positions, table_0, table_1, table_2, table_3, level_weights):
    lw16 = jnp.full((LANES,), -1e30, jnp.float32).at[:4].set(level_weights)
    return _call(positions, table_0, table_1, table_2, table_3, lw16)


# speedup vs baseline: 18.4827x; 1.0697x over previous
import functools

import jax
import jax.numpy as jnp
from jax import lax
from jax.experimental import pallas as pl
from jax.experimental.pallas import tpu as pltpu
from jax.experimental.pallas import tpu_sc as plsc

N = 1048576
NUM_LEVELS = 4
HASH_SIZES = (15, 17, 19, 21)
INV_CELL = (16.0, 64.0, 256.0, 1024.0)
C1 = 73856093
C2 = 19349663

NC = 2
NS = 16
NW = NC * NS
LANES = 16

B = 512
G = B // 128
PER_TILE = N // NW
CHUNKS = PER_TILE // B


def _full(v):
    return jnp.full((LANES,), v, jnp.int32)


def _body(pos_hbm, t0, t1, t2, t3, lw_hbm, out_hbm,
          pos_v, idx_v, wx_v, wy_v, rows_v, out_v, lw_v, sem):
    tables = (t0, t1, t2, t3)
    wid = lax.axis_index("s") * NC + lax.axis_index("c")
    tile_base = wid * PER_TILE

    pltpu.sync_copy(lw_hbm, lw_v)
    w = lw_v[...]
    w0 = plsc.load_gather(lw_v, [_full(0)])
    w1 = plsc.load_gather(lw_v, [_full(1)])
    w2 = plsc.load_gather(lw_v, [_full(2)])
    w3 = plsc.load_gather(lw_v, [_full(3)])
    m = jnp.maximum(jnp.maximum(w0, w1), jnp.maximum(w2, w3))
    lw_v[...] = jnp.exp(w - m)
    e = lw_v[...]
    e0 = plsc.load_gather(lw_v, [_full(0)])
    e1 = plsc.load_gather(lw_v, [_full(1)])
    e2 = plsc.load_gather(lw_v, [_full(2)])
    e3 = plsc.load_gather(lw_v, [_full(3)])
    lw_v[...] = e / ((e0 + e1) + (e2 + e3))

    iota = lax.iota(jnp.int32, LANES)

    def chunk_body(ci, carry):
        gbase = tile_base + ci * B
        pltpu.sync_copy(pos_hbm.at[pl.ds(gbase, B)], pos_v)

        def pa_body(j, c):
            r16 = pl.multiple_of(j * LANES, LANES) + iota
            px = plsc.load_gather(pos_v, [r16, _full(0)])
            py = plsc.load_gather(pos_v, [r16, _full(1)])
            gvec = jnp.full((LANES,), j >> 3, jnp.int32)
            colv = (j & 7) * LANES + iota
            for l in range(NUM_LEVELS):
                inv = INV_CELL[l]
                mask = (1 << HASH_SIZES[l]) - 1
                sx = px * inv
                sy = py * inv
                ix = sx.astype(jnp.int32)
                iy = sy.astype(jnp.int32)
                wx = sx - ix.astype(jnp.float32)
                wy = sy - iy.astype(jnp.float32)
                off = pl.multiple_of(j * LANES, LANES)
                wx_v[l, pl.ds(off, LANES)] = wx
                wy_v[l, pl.ds(off, LANES)] = wy
                hx0 = ix * C1
                hx1 = hx0 + C1
                hy0 = iy * C2
                hy1 = hy0 + C2
                hs = (hx0 ^ hy0, hx1 ^ hy0, hx0 ^ hy1, hx1 ^ hy1)
                for cidx in range(4):
                    cc = l * 4 + cidx
                    plsc.store_scatter(idx_v, [_full(cc), gvec, colv],
                                       hs[cidx] & mask)
            return c

        lax.fori_loop(0, B // LANES, pa_body, 0)

        copies = []
        for cc in range(4 * NUM_LEVELS):
            tbl = tables[cc // 4]
            for g in range(G):
                copies.append(pltpu.async_copy(
                    tbl.at[idx_v.at[cc, g]],
                    rows_v.at[cc, pl.ds(g * 128, 128)],
                    sem))
        for cp in copies:
            cp.wait()

        pair = iota >> 1
        parity = iota & 1

        def pc_body(i, c):
            rowv = i * 8 + pair
            pcol = (i & 15) * 8 + pair
            rbase = (i >> 4) * 8
            for l in range(NUM_LEVELS):
                base_c = l * 4
                f00 = plsc.load_gather(rows_v, [_full(base_c), rowv, parity])
                f10 = plsc.load_gather(rows_v, [_full(base_c + 1), rowv, parity])
                f01 = plsc.load_gather(rows_v, [_full(base_c + 2), rowv, parity])
                f11 = plsc.load_gather(rows_v, [_full(base_c + 3), rowv, parity])
                wxd = plsc.load_gather(wx_v, [_full(l), rowv])
                wyd = plsc.load_gather(wy_v, [_full(l), rowv])
                wl = plsc.load_gather(lw_v, [_full(l)])
                a = f00 + wxd * (f10 - f00)
                b = f01 + wxd * (f11 - f01)
                res = a + wyd * (b - a)
                orow = jnp.full((LANES,), rbase + 2 * l, jnp.int32) + parity
                plsc.store_scatter(out_v, [orow, pcol], wl * res)
            return c

        lax.fori_loop(0, B // 8, pc_body, 0)

        pltpu.sync_copy(out_v,
                        out_hbm.at[pl.ds((gbase // 128) * 8, (B // 128) * 8)])
        return carry

    lax.fori_loop(0, CHUNKS, chunk_body, 0)


_mesh = plsc.VectorSubcoreMesh(core_axis_name="c", subcore_axis_name="s")

_call = functools.partial(
    pl.kernel,
    mesh=_mesh,
    compiler_params=pltpu.CompilerParams(
        needs_layout_passes=False, use_tc_tiling_on_sc=False),
    out_type=jax.ShapeDtypeStruct((N // 128 * 8, 128), jnp.float32),
    scratch_types=[
        pltpu.VMEM((B, 2), jnp.float32),
        pltpu.VMEM((4 * NUM_LEVELS, G, 128), jnp.int32),
        pltpu.VMEM((NUM_LEVELS, B), jnp.float32),
        pltpu.VMEM((NUM_LEVELS, B), jnp.float32),
        pltpu.VMEM((4 * NUM_LEVELS, B, 2), jnp.float32),
        pltpu.VMEM((B // 128 * 8, 128), jnp.float32),
        pltpu.VMEM((LANES,), jnp.float32),
        pltpu.SemaphoreType.DMA,
    ],
)(_body)


@jax.jit
def kernel(positions, table_0, table_1, table_2, table_3, level_weights):
    lw16 = jnp.full((LANES,), -1e30, jnp.float32).at[:4].set(level_weights)
    out3 = _call(positions, table_0, table_1, table_2, table_3, lw16)
    return out3.reshape(N // 128, 8, 128).transpose(0, 2, 1).reshape(N, 8)
